# baseline (speedup 1.0000x reference)
import jax
import jax.numpy as jnp
from jax import lax
from jax.experimental import pallas as pl
from jax.experimental.pallas import tpu as pltpu
from jax.experimental.pallas import tpu_sc as plsc

EMBEDDING_DIM = 32
NUM_EMBEDDINGS = 8192
COMMITMENT_COST = 0.25

TN = 256

_NC, _NS, _L = 2, 16, 16
_NW = _NC * _NS


def _pass1_body(x_ref, e_ref, d_ref, idx_ref):
    x = x_ref[...]
    e = e_ref[...]
    x2 = jnp.sum(x * x, axis=1, keepdims=True)
    e2 = jnp.sum(e * e, axis=0, keepdims=True)
    xe = jnp.dot(x, e, preferred_element_type=jnp.float32)
    d = x2 - 2.0 * xe + e2
    d_ref[...] = d
    idx_ref[...] = jnp.argmin(d, axis=1)[:, None]


def _pass2_body(idx_ref, enc_ref, perp_ref, counts_ref):
    n = pl.program_id(0)
    nn = pl.num_programs(0)
    K = enc_ref.shape[1]
    lanes = jax.lax.broadcasted_iota(jnp.int32, (TN, K), 1)
    enc = (lanes == idx_ref[...]).astype(jnp.float32)
    enc_ref[...] = enc
    colsum = jnp.sum(enc, axis=0, keepdims=True)

    @pl.when(n == 0)
    def _init():
        counts_ref[...] = colsum

    @pl.when(n > 0)
    def _acc():
        counts_ref[...] += colsum

    @pl.when(n == nn - 1)
    def _final():
        rows = jnp.float32(TN) * nn
        avg = counts_ref[...] / rows
        ent = jnp.sum(avg * jnp.log(avg + 1e-10))
        perp_ref[...] = jnp.exp(-ent).reshape(1, 1)


def _sc_body(wt_ref, idx_ref, x_ref, q_ref, losspw_ref,
             idx_v, rows_v, x_v, acc_v, sem):
    bpw = idx_v.shape[0]
    wid = lax.axis_index("s") * _NC + lax.axis_index("c")
    base = wid * bpw

    pltpu.sync_copy(idx_ref.at[pl.ds(base, bpw)], idx_v)
    gather = pltpu.async_copy(wt_ref.at[idx_v], rows_v, sem)
    pltpu.sync_copy(x_ref.at[pl.ds(base, bpw)], x_v)
    gather.wait()
    pltpu.sync_copy(rows_v, q_ref.at[pl.ds(base, bpw)])

    def loss_row(i, acc):
        a = rows_v[i, pl.ds(0, _L)] - x_v[i, pl.ds(0, _L)]
        b = rows_v[i, pl.ds(_L, _L)] - x_v[i, pl.ds(_L, _L)]
        return acc + a * a + b * b
    acc_v[...] = lax.fori_loop(0, bpw, loss_row,
                               jnp.zeros((_L,), jnp.float32))
    pltpu.sync_copy(acc_v, losspw_ref.at[pl.ds(wid * _L, _L)])


def _finish_body(losspw_ref, loss_ref):
    total_el = jnp.float32(16384) * jnp.float32(EMBEDDING_DIM)
    loss_ref[...] = (jnp.sum(losspw_ref[...])
                     * (COMMITMENT_COST / total_el)).reshape(1, 1)


def kernel(inputs, embeddings, is_training):
    del is_training
    D = embeddings.shape[0]
    K = embeddings.shape[1]
    flat = jnp.reshape(inputs, (-1, D))
    N = flat.shape[0]
    nn = N // TN
    bpw = N // _NW

    distances, idx2d = pl.pallas_call(
        _pass1_body,
        grid=(nn,),
        in_specs=[
            pl.BlockSpec((TN, D), lambda n: (n, 0)),
            pl.BlockSpec((D, K), lambda n: (0, 0)),
        ],
        out_specs=[
            pl.BlockSpec((TN, K), lambda n: (n, 0)),
            pl.BlockSpec((TN, 1), lambda n: (n, 0)),
        ],
        out_shape=[
            jax.ShapeDtypeStruct((N, K), jnp.float32),
            jax.ShapeDtypeStruct((N, 1), jnp.int32),
        ],
    )(flat, embeddings)

    sc_quantize = pl.kernel(
        _sc_body,
        mesh=plsc.VectorSubcoreMesh(core_axis_name="c", subcore_axis_name="s",
                                    num_cores=_NC, num_subcores=_NS),
        compiler_params=pltpu.CompilerParams(use_tc_tiling_on_sc=False),
        out_type=[
            jax.ShapeDtypeStruct((N, D), jnp.float32),
            jax.ShapeDtypeStruct((_NW * _L,), jnp.float32),
        ],
        scratch_types=[
            pltpu.VMEM((bpw,), jnp.int32),
            pltpu.VMEM((bpw, D), jnp.float32),
            pltpu.VMEM((bpw, D), jnp.float32),
            pltpu.VMEM((_L,), jnp.float32),
            pltpu.SemaphoreType.DMA,
        ],
    )
    wt = embeddings.T
    idx_flat = jnp.reshape(idx2d, (N,))
    quant, loss_pw = sc_quantize(wt, idx_flat, flat)

    encodings, perp11 = pl.pallas_call(
        _pass2_body,
        grid=(nn,),
        in_specs=[pl.BlockSpec((TN, 1), lambda n: (n, 0))],
        out_specs=[
            pl.BlockSpec((TN, K), lambda n: (n, 0)),
            pl.BlockSpec((1, 1), lambda n: (0, 0)),
        ],
        out_shape=[
            jax.ShapeDtypeStruct((N, K), jnp.float32),
            jax.ShapeDtypeStruct((1, 1), jnp.float32),
        ],
        scratch_shapes=[pltpu.VMEM((1, K), jnp.float32)],
    )(idx2d)

    loss11, = pl.pallas_call(
        _finish_body,
        grid=(1,),
        in_specs=[pl.BlockSpec((_NW, _L), lambda i: (0, 0))],
        out_specs=[pl.BlockSpec((1, 1), lambda i: (0, 0))],
        out_shape=[jax.ShapeDtypeStruct((1, 1), jnp.float32)],
    )(jnp.reshape(loss_pw, (_NW, _L)))

    quantized = jnp.reshape(quant, inputs.shape)
    encoding_indices = jnp.reshape(idx2d, inputs.shape[:-1])
    loss = loss11[0, 0]
    perplexity = perp11[0, 0]
    return (quantized, loss, perplexity, encodings, encoding_indices, distances)


# speedup vs baseline: 1.0363x; 1.0363x over previous
import jax
import jax.numpy as jnp
from jax import lax
from jax.experimental import pallas as pl
from jax.experimental.pallas import tpu as pltpu
from jax.experimental.pallas import tpu_sc as plsc

EMBEDDING_DIM = 32
NUM_EMBEDDINGS = 8192
COMMITMENT_COST = 0.25

TN = 256

_NC, _NS, _L = 2, 16, 16
_NW = _NC * _NS


def _tc_body(x_ref, e_ref, d_ref, idx_ref, enc_ref, perp_ref, counts_ref):
    n = pl.program_id(0)
    nn = pl.num_programs(0)
    K = e_ref.shape[1]

    x = x_ref[...]
    e = e_ref[...]
    x2 = jnp.sum(x * x, axis=1, keepdims=True)
    e2 = jnp.sum(e * e, axis=0, keepdims=True)
    xe = jnp.dot(x, e, preferred_element_type=jnp.float32)
    d = x2 - 2.0 * xe + e2
    d_ref[...] = d

    idx = jnp.argmin(d, axis=1)[:, None]
    idx_ref[...] = idx

    lanes = jax.lax.broadcasted_iota(jnp.int32, (TN, K), 1)
    enc = (lanes == idx).astype(jnp.float32)
    enc_ref[...] = enc
    colsum = jnp.sum(enc, axis=0, keepdims=True)

    @pl.when(n == 0)
    def _init():
        counts_ref[...] = colsum

    @pl.when(n > 0)
    def _acc():
        counts_ref[...] += colsum

    @pl.when(n == nn - 1)
    def _final():
        rows = jnp.float32(TN) * nn
        avg = counts_ref[...] / rows
        ent = jnp.sum(avg * jnp.log(avg + 1e-10))
        perp_ref[...] = jnp.exp(-ent).reshape(1, 1)


def _sc_body(wt_ref, idx_ref, x_ref, q_ref, losspw_ref,
             idx_v, rows_v, x_v, acc_v, sem):
    bpw = idx_v.shape[0]
    wid = lax.axis_index("s") * _NC + lax.axis_index("c")
    base = wid * bpw

    pltpu.sync_copy(idx_ref.at[pl.ds(base, bpw)], idx_v)
    gather = pltpu.async_copy(wt_ref.at[idx_v], rows_v, sem)
    pltpu.sync_copy(x_ref.at[pl.ds(base, bpw)], x_v)
    gather.wait()
    pltpu.sync_copy(rows_v, q_ref.at[pl.ds(base, bpw)])

    def loss_row(i, acc):
        a = rows_v[i, pl.ds(0, _L)] - x_v[i, pl.ds(0, _L)]
        b = rows_v[i, pl.ds(_L, _L)] - x_v[i, pl.ds(_L, _L)]
        return acc + a * a + b * b
    acc_v[...] = lax.fori_loop(0, bpw, loss_row,
                               jnp.zeros((_L,), jnp.float32))
    pltpu.sync_copy(acc_v, losspw_ref.at[pl.ds(wid * _L, _L)])


def _finish_body(losspw_ref, loss_ref):
    total_el = jnp.float32(16384) * jnp.float32(EMBEDDING_DIM)
    loss_ref[...] = (jnp.sum(losspw_ref[...])
                     * (COMMITMENT_COST / total_el)).reshape(1, 1)


def kernel(inputs, embeddings, is_training):
    del is_training
    D = embeddings.shape[0]
    K = embeddings.shape[1]
    flat = jnp.reshape(inputs, (-1, D))
    N = flat.shape[0]
    nn = N // TN
    bpw = N // _NW

    distances, idx2d, encodings, perp11 = pl.pallas_call(
        _tc_body,
        grid=(nn,),
        in_specs=[
            pl.BlockSpec((TN, D), lambda n: (n, 0)),
            pl.BlockSpec((D, K), lambda n: (0, 0)),
        ],
        out_specs=[
            pl.BlockSpec((TN, K), lambda n: (n, 0)),
            pl.BlockSpec((TN, 1), lambda n: (n, 0)),
            pl.BlockSpec((TN, K), lambda n: (n, 0)),
            pl.BlockSpec((1, 1), lambda n: (0, 0)),
        ],
        out_shape=[
            jax.ShapeDtypeStruct((N, K), jnp.float32),
            jax.ShapeDtypeStruct((N, 1), jnp.int32),
            jax.ShapeDtypeStruct((N, K), jnp.float32),
            jax.ShapeDtypeStruct((1, 1), jnp.float32),
        ],
        scratch_shapes=[pltpu.VMEM((1, K), jnp.float32)],
    )(flat, embeddings)

    sc_quantize = pl.kernel(
        _sc_body,
        mesh=plsc.VectorSubcoreMesh(core_axis_name="c", subcore_axis_name="s",
                                    num_cores=_NC, num_subcores=_NS),
        compiler_params=pltpu.CompilerParams(use_tc_tiling_on_sc=False),
        out_type=[
            jax.ShapeDtypeStruct((N, D), jnp.float32),
            jax.ShapeDtypeStruct((_NW * _L,), jnp.float32),
        ],
        scratch_types=[
            pltpu.VMEM((bpw,), jnp.int32),
            pltpu.VMEM((bpw, D), jnp.float32),
            pltpu.VMEM((bpw, D), jnp.float32),
            pltpu.VMEM((_L,), jnp.float32),
            pltpu.SemaphoreType.DMA,
        ],
    )
    wt = embeddings.T
    idx_flat = jnp.reshape(idx2d, (N,))
    quant, loss_pw = sc_quantize(wt, idx_flat, flat)

    loss11, = pl.pallas_call(
        _finish_body,
        grid=(1,),
        in_specs=[pl.BlockSpec((_NW, _L), lambda i: (0, 0))],
        out_specs=[pl.BlockSpec((1, 1), lambda i: (0, 0))],
        out_shape=[jax.ShapeDtypeStruct((1, 1), jnp.float32)],
    )(jnp.reshape(loss_pw, (_NW, _L)))

    quantized = jnp.reshape(quant, inputs.shape)
    encoding_indices = jnp.reshape(idx2d, inputs.shape[:-1])
    loss = loss11[0, 0]
    perplexity = perp11[0, 0]
    return (quantized, loss, perplexity, encodings, encoding_indices, distances)
